# baseline (speedup 1.0000x reference)
import functools

import jax
import jax.numpy as jnp
from jax import lax
from jax.experimental import pallas as pl
from jax.experimental.pallas import tpu as pltpu
from jax.experimental.pallas import tpu_sc as plsc

V = 1000000
D = 64
B = 16384

NC = 2
NS = 16
L = 16
NW = NC * NS
BPW = B // NW
NCHUNK = 4
CHUNK = BPW // NCHUNK
GROUPS_PER_CHUNK = CHUNK // L


def _body(e0_hbm, e1_hbm, ids0_hbm, ids1_hbm, out_hbm,
          idx0_v, idx1_v, rows0_v, rows1_v, out_v,
          sem0, sem1, sem2, sem3):
    sems = [sem0, sem1, sem2, sem3]
    wid = lax.axis_index("s") * NC + lax.axis_index("c")
    base = wid * BPW

    for c in range(NCHUNK):
        pltpu.sync_copy(ids0_hbm.at[pl.ds(base + c * CHUNK, CHUNK)],
                        idx0_v.at[c])
        pltpu.sync_copy(ids1_hbm.at[pl.ds(base + c * CHUNK, CHUNK)],
                        idx1_v.at[c])

    copies = []
    for c in range(NCHUNK):
        rsl = pl.ds(c * CHUNK, CHUNK)
        copies.append(
            pltpu.async_copy(e0_hbm.at[idx0_v.at[c]], rows0_v.at[rsl], sems[c]))
        copies.append(
            pltpu.async_copy(e1_hbm.at[idx1_v.at[c]], rows1_v.at[rsl], sems[c]))

    lane = lax.iota(jnp.int32, L)

    for c in range(NCHUNK):
        copies[2 * c].wait()
        copies[2 * c + 1].wait()

        def group_body(g, _, c=c):
            row_base = (c * GROUPS_PER_CHUNK + g) * L
            acc = jnp.zeros((L,), jnp.float32)
            for j in range(L):
                r = row_base + j
                s = jnp.zeros((L,), jnp.float32)
                for k in range(D // L):
                    ksl = pl.ds(k * L, L)
                    s = s + rows0_v[r, ksl] * rows1_v[r, ksl]
                acc = jnp.where(lane == j, jnp.sum(s), acc)
            out_v[pl.ds(row_base, L)] = acc
            return _

        lax.fori_loop(0, GROUPS_PER_CHUNK, group_body, None)

    pltpu.sync_copy(out_v, out_hbm.at[pl.ds(base, BPW)])


@jax.jit
def _run(E0, E1, ids0, ids1):
    mesh = plsc.VectorSubcoreMesh(core_axis_name="c", subcore_axis_name="s")
    kfn = pl.kernel(
        _body,
        out_type=jax.ShapeDtypeStruct((B,), jnp.float32),
        mesh=mesh,
        compiler_params=pltpu.CompilerParams(
            needs_layout_passes=False, use_tc_tiling_on_sc=False),
        scratch_types=[
            pltpu.VMEM((NCHUNK, CHUNK), jnp.int32),
            pltpu.VMEM((NCHUNK, CHUNK), jnp.int32),
            pltpu.VMEM((BPW, D), jnp.float32),
            pltpu.VMEM((BPW, D), jnp.float32),
            pltpu.VMEM((BPW,), jnp.float32),
            pltpu.SemaphoreType.DMA,
            pltpu.SemaphoreType.DMA,
            pltpu.SemaphoreType.DMA,
            pltpu.SemaphoreType.DMA,
        ],
    )
    return kfn(E0, E1, ids0, ids1)


def kernel(E0, E1, ids0, ids1):
    return _run(E0, E1, ids0, ids1).reshape(B, 1)


# speedup vs baseline: 1.0021x; 1.0021x over previous
import functools

import jax
import jax.numpy as jnp
from jax import lax
from jax.experimental import pallas as pl
from jax.experimental.pallas import tpu as pltpu
from jax.experimental.pallas import tpu_sc as plsc

V = 1000000
D = 64
B = 16384

NC = 2
NS = 16
L = 16
NW = NC * NS
BPW = B // NW
NCHUNK = 4
CHUNK = BPW // NCHUNK
GROUPS_PER_CHUNK = CHUNK // L


def _body(e0_hbm, e1_hbm, ids0_hbm, ids1_hbm, out_hbm,
          idx0_v, idx1_v, rows0_v, rows1_v, out_v,
          sem0, sem1, sem2, sem3):
    sems = [sem0, sem1, sem2, sem3]
    wid = lax.axis_index("s") * NC + lax.axis_index("c")
    base = wid * BPW

    idx_copies = []
    for c in range(NCHUNK):
        idx_copies.append(pltpu.async_copy(
            ids0_hbm.at[pl.ds(base + c * CHUNK, CHUNK)], idx0_v.at[c], sem0))
        idx_copies.append(pltpu.async_copy(
            ids1_hbm.at[pl.ds(base + c * CHUNK, CHUNK)], idx1_v.at[c], sem0))
    for cp in idx_copies:
        cp.wait()

    copies = []
    for c in range(NCHUNK):
        rsl = pl.ds(c * CHUNK, CHUNK)
        copies.append(
            pltpu.async_copy(e0_hbm.at[idx0_v.at[c]], rows0_v.at[rsl], sems[c]))
        copies.append(
            pltpu.async_copy(e1_hbm.at[idx1_v.at[c]], rows1_v.at[rsl], sems[c]))

    lane = lax.iota(jnp.int32, L)

    for c in range(NCHUNK):
        copies[2 * c].wait()
        copies[2 * c + 1].wait()

        def group_body(g, _, c=c):
            row_base = (c * GROUPS_PER_CHUNK + g) * L
            rows = row_base + lane
            col = lane
            acc = jnp.zeros((L,), jnp.float32)
            for d in range(D):
                v0 = plsc.load_gather(rows0_v, [rows, col])
                v1 = plsc.load_gather(rows1_v, [rows, col])
                acc = acc + v0 * v1
                if d + 1 < D:
                    col = col + jnp.where(lane == D - 1 - d, 1 - D, 1)
            out_v[pl.ds(row_base, L)] = acc
            return _

        lax.fori_loop(0, GROUPS_PER_CHUNK, group_body, None)

    pltpu.sync_copy(out_v, out_hbm.at[pl.ds(base, BPW)])


@jax.jit
def _run(E0, E1, ids0, ids1):
    mesh = plsc.VectorSubcoreMesh(core_axis_name="c", subcore_axis_name="s")
    kfn = pl.kernel(
        _body,
        out_type=jax.ShapeDtypeStruct((B,), jnp.float32),
        mesh=mesh,
        compiler_params=pltpu.CompilerParams(
            needs_layout_passes=False, use_tc_tiling_on_sc=False),
        scratch_types=[
            pltpu.VMEM((NCHUNK, CHUNK), jnp.int32),
            pltpu.VMEM((NCHUNK, CHUNK), jnp.int32),
            pltpu.VMEM((BPW, D), jnp.float32),
            pltpu.VMEM((BPW, D), jnp.float32),
            pltpu.VMEM((BPW,), jnp.float32),
            pltpu.SemaphoreType.DMA,
            pltpu.SemaphoreType.DMA,
            pltpu.SemaphoreType.DMA,
            pltpu.SemaphoreType.DMA,
        ],
    )
    return kfn(E0, E1, ids0, ids1)


def kernel(E0, E1, ids0, ids1):
    return _run(E0, E1, ids0, ids1).reshape(B, 1)


# speedup vs baseline: 2.1829x; 2.1782x over previous
import functools

import jax
import jax.numpy as jnp
from jax import lax
from jax.experimental import pallas as pl
from jax.experimental.pallas import tpu as pltpu
from jax.experimental.pallas import tpu_sc as plsc

V = 1000000
D = 64
B = 16384

NC = 2
NS = 16
L = 16
NW = NC * NS
BPW = B // NW
NCHUNK = 16
CHUNK = BPW // NCHUNK
GPC = CHUNK // L
SUB = 8


def _body(e0_hbm, e1_hbm, ids0_hbm, ids1_hbm, out_hbm,
          idx0_v, idx1_v, rows0_v, rows1_v, out_v,
          sem_idx, sem0, sem1):
    wid = lax.axis_index("s") * NC + lax.axis_index("c")
    base = wid * BPW

    ci0 = pltpu.async_copy(ids0_hbm.at[pl.ds(base, BPW)], idx0_v, sem_idx)
    ci1 = pltpu.async_copy(ids1_hbm.at[pl.ds(base, BPW)], idx1_v, sem_idx)
    ci0.wait()
    ci1.wait()

    lane = lax.iota(jnp.int32, L)

    def chunk_body(c, _):
        cbase = c * CHUNK
        for g in range(GPC):
            idv0 = idx0_v[pl.ds(cbase + g * L, L)]
            idv1 = idx1_v[pl.ds(cbase + g * L, L)]
            for j in range(L):
                slot = g * L + j
                pltpu.async_copy(
                    e0_hbm.at[pl.ds(jnp.right_shift(idv0[j], 3), 1)],
                    rows0_v.at[pl.ds(slot, 1)], sem0)
                pltpu.async_copy(
                    e1_hbm.at[pl.ds(jnp.right_shift(idv1[j], 3), 1)],
                    rows1_v.at[pl.ds(slot, 1)], sem1)
        for _i in range(CHUNK):
            pltpu.make_async_copy(e0_hbm.at[pl.ds(0, 1)],
                                  rows0_v.at[pl.ds(0, 1)], sem0).wait()
            pltpu.make_async_copy(e1_hbm.at[pl.ds(0, 1)],
                                  rows1_v.at[pl.ds(0, 1)], sem1).wait()

        for g in range(GPC):
            row_base = cbase + g * L
            slot = g * L + lane
            sub0 = jnp.bitwise_and(idx0_v[pl.ds(row_base, L)], SUB - 1)
            sub1 = jnp.bitwise_and(idx1_v[pl.ds(row_base, L)], SUB - 1)
            col = lane
            acc = jnp.zeros((L,), jnp.float32)
            for d in range(D):
                v0 = plsc.load_gather(rows0_v, [slot, sub0, col])
                v1 = plsc.load_gather(rows1_v, [slot, sub1, col])
                acc = acc + v0 * v1
                if d + 1 < D:
                    col = col + jnp.where(lane == D - 1 - d, 1 - D, 1)
            out_v[pl.ds(row_base, L)] = acc
        return _

    lax.fori_loop(0, NCHUNK, chunk_body, None)

    pltpu.sync_copy(out_v, out_hbm.at[pl.ds(base, BPW)])


@jax.jit
def _run(E0, E1, ids0, ids1):
    mesh = plsc.VectorSubcoreMesh(core_axis_name="c", subcore_axis_name="s")
    kfn = pl.kernel(
        _body,
        out_type=jax.ShapeDtypeStruct((B,), jnp.float32),
        mesh=mesh,
        compiler_params=pltpu.CompilerParams(needs_layout_passes=False),
        scratch_types=[
            pltpu.VMEM((BPW,), jnp.int32),
            pltpu.VMEM((BPW,), jnp.int32),
            pltpu.VMEM((CHUNK, SUB, D), jnp.float32),
            pltpu.VMEM((CHUNK, SUB, D), jnp.float32),
            pltpu.VMEM((BPW,), jnp.float32),
            pltpu.SemaphoreType.DMA,
            pltpu.SemaphoreType.DMA,
            pltpu.SemaphoreType.DMA,
        ],
    )
    return kfn(E0.reshape(V // SUB, SUB, D), E1.reshape(V // SUB, SUB, D),
               ids0, ids1)


def kernel(E0, E1, ids0, ids1):
    return _run(E0, E1, ids0, ids1).reshape(B, 1)
